# baseline (speedup 1.0000x reference)
import jax
import jax.numpy as jnp
from jax.experimental import pallas as pl

BLOCK_NUM = 256


def _sgc_kernel(rm_ref, adj_ref, out_ref):
    adj = adj_ref[0]
    rm = rm_ref[0]
    a2 = jnp.dot(adj, adj, preferred_element_type=jnp.float32)
    t = jnp.dot(a2, rm, preferred_element_type=jnp.float32)
    full = jnp.dot(a2, t, preferred_element_type=jnp.float32)
    diag = jnp.sum(a2 * a2.T, axis=1, keepdims=True)
    out_ref[0] = full - diag * rm


def kernel(regional_means, adj):
    b, n, d = regional_means.shape
    return pl.pallas_call(
        _sgc_kernel,
        grid=(b,),
        in_specs=[
            pl.BlockSpec((1, n, d), lambda i: (i, 0, 0)),
            pl.BlockSpec((1, n, n), lambda i: (i, 0, 0)),
        ],
        out_specs=pl.BlockSpec((1, n, d), lambda i: (i, 0, 0)),
        out_shape=jax.ShapeDtypeStruct((b, n, d), jnp.float32),
    )(regional_means, adj)


# speedup vs baseline: 1.2954x; 1.2954x over previous
import jax
import jax.numpy as jnp
from jax.experimental import pallas as pl

BLOCK_NUM = 256


BB = 2


def _sgc_kernel(rm_ref, adj_ref, out_ref):
    for k in range(BB):
        adj = adj_ref[k]
        rm = rm_ref[k]
        a2 = jnp.dot(adj, adj, preferred_element_type=jnp.float32)
        t = jnp.dot(a2, rm, preferred_element_type=jnp.float32)
        full = jnp.dot(a2, t, preferred_element_type=jnp.float32)
        diag = jnp.sum(a2 * a2.T, axis=1, keepdims=True)
        out_ref[k] = full - diag * rm


def kernel(regional_means, adj):
    b, n, d = regional_means.shape
    return pl.pallas_call(
        _sgc_kernel,
        grid=(b // BB,),
        in_specs=[
            pl.BlockSpec((BB, n, d), lambda i: (i, 0, 0)),
            pl.BlockSpec((BB, n, n), lambda i: (i, 0, 0)),
        ],
        out_specs=pl.BlockSpec((BB, n, d), lambda i: (i, 0, 0)),
        out_shape=jax.ShapeDtypeStruct((b, n, d), jnp.float32),
    )(regional_means, adj)


# speedup vs baseline: 1.4655x; 1.1313x over previous
import jax
import jax.numpy as jnp
from jax.experimental import pallas as pl

BLOCK_NUM = 256


BB = 4


def _sgc_kernel(rm_ref, adj_ref, out_ref):
    for k in range(BB):
        adj = adj_ref[k]
        rm = rm_ref[k]
        a2 = jnp.dot(adj, adj, preferred_element_type=jnp.float32)
        t = jnp.dot(a2, rm, preferred_element_type=jnp.float32)
        full = jnp.dot(a2, t, preferred_element_type=jnp.float32)
        diag = jnp.sum(a2 * a2.T, axis=1, keepdims=True)
        out_ref[k] = full - diag * rm


def kernel(regional_means, adj):
    b, n, d = regional_means.shape
    return pl.pallas_call(
        _sgc_kernel,
        grid=(b // BB,),
        in_specs=[
            pl.BlockSpec((BB, n, d), lambda i: (i, 0, 0)),
            pl.BlockSpec((BB, n, n), lambda i: (i, 0, 0)),
        ],
        out_specs=pl.BlockSpec((BB, n, d), lambda i: (i, 0, 0)),
        out_shape=jax.ShapeDtypeStruct((b, n, d), jnp.float32),
    )(regional_means, adj)


# speedup vs baseline: 1.5059x; 1.0276x over previous
import jax
import jax.numpy as jnp
from jax.experimental import pallas as pl

BLOCK_NUM = 256


BB = 8


def _sgc_kernel(rm_ref, adj_ref, out_ref):
    for k in range(BB):
        adj = adj_ref[k]
        rm = rm_ref[k]
        a2 = jnp.dot(adj, adj, preferred_element_type=jnp.float32)
        t = jnp.dot(a2, rm, preferred_element_type=jnp.float32)
        full = jnp.dot(a2, t, preferred_element_type=jnp.float32)
        diag = jnp.sum(a2 * a2.T, axis=1, keepdims=True)
        out_ref[k] = full - diag * rm


def kernel(regional_means, adj):
    b, n, d = regional_means.shape
    return pl.pallas_call(
        _sgc_kernel,
        grid=(b // BB,),
        in_specs=[
            pl.BlockSpec((BB, n, d), lambda i: (i, 0, 0)),
            pl.BlockSpec((BB, n, n), lambda i: (i, 0, 0)),
        ],
        out_specs=pl.BlockSpec((BB, n, d), lambda i: (i, 0, 0)),
        out_shape=jax.ShapeDtypeStruct((b, n, d), jnp.float32),
    )(regional_means, adj)
